# baseline (speedup 1.0000x reference)
import jax
import jax.numpy as jnp
from jax.experimental import pallas as pl
from jax.experimental.pallas import tpu as pltpu

BI = 512
NB = 8
NRING = 3


def _scalar_rows(xp, alar, recip):
    els = jnp.dot(xp, alar, preferred_element_type=jnp.float32)
    el = els[:, 0:1] * recip
    er = els[:, 1:2] * recip
    return el, er


def _adj_copy(adj_hbm, ring_s, sems, blk):
    return pltpu.make_async_copy(
        adj_hbm.at[pl.ds(blk * BI, BI), :],
        ring_s.at[jax.lax.rem(blk, NRING)],
        sems.at[jax.lax.rem(blk, NRING)])


def _body(x_ref, adj_hbm, W1_ref, alar1_ref, b1r_ref,
          W2_ref, alar2_ref, b2r_ref, out_ref,
          ring_s, adj8_s, xaug1_s, cols1_s, er1r_s, F1r_s, Fs1r_s,
          xaug2_s, cols2_s, er2r_s, F2r_s, Fs2r_s, sems):
    s = pl.program_id(0)
    N = adj8_s.shape[1]
    D1 = W1_ref.shape[1]
    D2 = W2_ref.shape[1]

    @pl.when(s == 0)
    def _proj():
        _adj_copy(adj_hbm, ring_s, sems, 0).start()
        _adj_copy(adj_hbm, ring_s, sems, 1).start()
        xv = jnp.dot(x_ref[...].astype(jnp.bfloat16),
                     W1_ref[...].astype(jnp.bfloat16),
                     preferred_element_type=jnp.float32)
        lane = jax.lax.broadcasted_iota(jnp.int32, (N, D1), 1)
        tail = jnp.where(lane == 0, 1.0, 0.0).astype(jnp.bfloat16)
        xaug1_s[...] = jnp.concatenate(
            [xv.astype(jnp.bfloat16), tail], axis=1)
        el = jnp.dot(xv, alar1_ref[:, 0:1],
                     preferred_element_type=jnp.float32)
        er_row = jax.lax.dot_general(
            alar1_ref[:, 1:2], xv, (((0,), (1,)), ((), ())),
            preferred_element_type=jnp.float32)
        cols1_s[...] = jnp.concatenate(
            [-el, jnp.exp(0.8 * el), jnp.zeros((N, 126), jnp.float32)],
            axis=1).astype(jnp.bfloat16)
        er1r_s[...] = er_row.astype(jnp.bfloat16)
        F1r_s[...] = jnp.exp(er_row).astype(jnp.bfloat16)
        Fs1r_s[...] = jnp.exp(0.2 * er_row).astype(jnp.bfloat16)

    @pl.when(jnp.logical_and(s >= 1, s < 1 + NB))
    def _att1():
        i = s - 1

        @pl.when(i + 2 < NB)
        def _prefetch():
            _adj_copy(adj_hbm, ring_s, sems, i + 2).start()

        _adj_copy(adj_hbm, ring_s, sems, i).wait()
        adj = ring_s[jax.lax.rem(i, NRING)]
        rows = pl.ds(i * BI, BI)
        adjb = adj.astype(jnp.bfloat16)
        adj8_s[rows, :] = adjb.astype(jnp.int8)
        c1 = cols1_s[rows, :]
        cond = er1r_s[...] > c1[:, 0:1]
        t = c1[:, 1:2] * F1r_s[...]
        B = jnp.where(cond, t, Fs1r_s[...]) * adjb
        numaug = jnp.dot(B, xaug1_s[...], preferred_element_type=jnp.float32)
        num = numaug[:, :D1]
        denom = numaug[:, D1:D1 + 1]
        recip = 1.0 / jnp.maximum(denom, 1e-12)
        hp = jnp.maximum(num + denom * b1r_ref[...], 0.0)
        x2p = jnp.dot(hp, W2_ref[...], preferred_element_type=jnp.float32)
        lane = jax.lax.broadcasted_iota(jnp.int32, (BI, D2), 1)
        tail = jnp.where(lane == 0, denom, 0.0).astype(jnp.bfloat16)
        xaug2_s[rows, :] = jnp.concatenate(
            [x2p.astype(jnp.bfloat16), tail], axis=1)
        el2 = jnp.dot(x2p, alar2_ref[:, 0:1],
                      preferred_element_type=jnp.float32) * recip
        cols2_s[rows, :] = jnp.concatenate(
            [-el2, jnp.exp(0.8 * el2), jnp.zeros((BI, 126), jnp.float32)],
            axis=1).astype(jnp.bfloat16)
        denom_row = jax.lax.dot_general(
            jnp.ones((1, N), jnp.bfloat16), B, (((1,), (1,)), ((), ())),
            preferred_element_type=jnp.float32)
        recip_row = 1.0 / jnp.maximum(denom_row, 1e-12)
        er2_row = jax.lax.dot_general(
            alar2_ref[:, 1:2], x2p, (((0,), (1,)), ((), ())),
            preferred_element_type=jnp.float32) * recip_row
        cols = pl.ds(i * BI, BI)
        er2r_s[:, cols] = er2_row.astype(jnp.bfloat16)
        F2r_s[:, cols] = (jnp.exp(er2_row) * recip_row).astype(jnp.bfloat16)
        Fs2r_s[:, cols] = (jnp.exp(0.2 * er2_row) * recip_row).astype(
            jnp.bfloat16)

    @pl.when(s >= 1 + NB)
    def _att2():
        i = s - 1 - NB
        rows = pl.ds(i * BI, BI)
        adjb = adj8_s[rows, :].astype(jnp.bfloat16)
        c2 = cols2_s[rows, :]
        cond = er2r_s[...] > c2[:, 0:1]
        t = c2[:, 1:2] * F2r_s[...]
        B = jnp.where(cond, t, Fs2r_s[...]) * adjb
        numaug = jnp.dot(B, xaug2_s[...], preferred_element_type=jnp.float32)
        num = numaug[:, :D2]
        d2 = numaug[:, D2:D2 + 1]
        h = num / jnp.maximum(d2, 1e-12) + b2r_ref[...]
        m = jnp.max(h, axis=1, keepdims=True)
        hs = h - m
        lse = jnp.log(jnp.sum(jnp.exp(hs), axis=1, keepdims=True))
        out_ref[...] = hs - lse


def kernel(x, adj, W1, al1, ar1, b1, W2, al2, ar2, b2):
    N, K = x.shape
    D1 = W1.shape[1]
    D2 = W2.shape[1]
    bf = jnp.bfloat16

    return pl.pallas_call(
        _body,
        grid=(1 + 2 * NB,),
        in_specs=[
            pl.BlockSpec((N, K), lambda s: (0, 0)),
            pl.BlockSpec(memory_space=pltpu.MemorySpace.HBM),
            pl.BlockSpec((K, D1), lambda s: (0, 0)),
            pl.BlockSpec((D1, 2), lambda s: (0, 0)),
            pl.BlockSpec((1, D1), lambda s: (0, 0)),
            pl.BlockSpec((D1, D2), lambda s: (0, 0)),
            pl.BlockSpec((D2, 2), lambda s: (0, 0)),
            pl.BlockSpec((1, D2), lambda s: (0, 0)),
        ],
        out_specs=pl.BlockSpec(
            (BI, D2), lambda s: (jnp.clip(s - 1 - NB, 0, NB - 1), 0)),
        out_shape=jax.ShapeDtypeStruct((N, D2), jnp.float32),
        scratch_shapes=[
            pltpu.VMEM((NRING, BI, N), jnp.float32),
            pltpu.VMEM((N, N), jnp.int8),
            pltpu.VMEM((N, 2 * D1), bf),
            pltpu.VMEM((N, 128), bf),
            pltpu.VMEM((1, N), bf),
            pltpu.VMEM((1, N), bf),
            pltpu.VMEM((1, N), bf),
            pltpu.VMEM((N, 2 * D2), bf),
            pltpu.VMEM((N, 128), bf),
            pltpu.VMEM((1, N), bf),
            pltpu.VMEM((1, N), bf),
            pltpu.VMEM((1, N), bf),
            pltpu.SemaphoreType.DMA((NRING,)),
        ],
        compiler_params=pltpu.CompilerParams(
            dimension_semantics=("arbitrary",)),
    )(x, adj, W1, jnp.concatenate([al1, ar1], axis=1),
      b1.reshape(1, D1), W2, jnp.concatenate([al2, ar2], axis=1),
      b2.reshape(1, D2))


# speedup vs baseline: 1.3402x; 1.3402x over previous
import jax
import jax.numpy as jnp
from jax.experimental import pallas as pl
from jax.experimental.pallas import tpu as pltpu

BI = 512
NB = 8
NRING = 3


def _scalar_rows(xp, alar, recip):
    els = jnp.dot(xp, alar, preferred_element_type=jnp.float32)
    el = els[:, 0:1] * recip
    er = els[:, 1:2] * recip
    return el, er


def _adj_copy(adj_hbm, ring_s, sems, blk):
    return pltpu.make_async_copy(
        adj_hbm.at[pl.ds(blk * BI, BI), :],
        ring_s.at[jax.lax.rem(blk, NRING)],
        sems.at[jax.lax.rem(blk, NRING)])


def _body(x_ref, adj_hbm, W1_ref, alar1_ref, b1r_ref,
          W2_ref, alar2_ref, b2r_ref, out_ref,
          ring_s, adj8_s, xaug1_s, cols1_s, er1r_s, F1r_s, Fs1r_s,
          xaug2_s, cols2_s, er2r_s, F2r_s, Fs2r_s, sems):
    s = pl.program_id(0)
    N = adj8_s.shape[1]
    D1 = W1_ref.shape[1]
    D2 = W2_ref.shape[1]

    @pl.when(s == 0)
    def _proj():
        _adj_copy(adj_hbm, ring_s, sems, 0).start()
        _adj_copy(adj_hbm, ring_s, sems, 1).start()
        xv = jnp.dot(x_ref[...].astype(jnp.bfloat16),
                     W1_ref[...].astype(jnp.bfloat16),
                     preferred_element_type=jnp.float32)
        lane = jax.lax.broadcasted_iota(jnp.int32, (N, D1), 1)
        tail = jnp.where(lane == 0, 1.0, 0.0).astype(jnp.bfloat16)
        xaug1_s[...] = jnp.concatenate(
            [xv.astype(jnp.bfloat16), tail], axis=1)
        el, er = _scalar_rows(xv, alar1_ref[...], 1.0)
        cols1_s[...] = jnp.concatenate(
            [-el, jnp.exp(0.8 * el), jnp.zeros((N, 126), jnp.float32)],
            axis=1).astype(jnp.bfloat16)
        er1r_s[...] = er.astype(jnp.bfloat16).reshape(1, N)
        F1r_s[...] = jnp.exp(er).astype(jnp.bfloat16).reshape(1, N)
        Fs1r_s[...] = jnp.exp(0.2 * er).astype(jnp.bfloat16).reshape(1, N)

    @pl.when(jnp.logical_and(s >= 1, s < 1 + NB))
    def _att1():
        i = s - 1

        @pl.when(i + 2 < NB)
        def _prefetch():
            _adj_copy(adj_hbm, ring_s, sems, i + 2).start()

        _adj_copy(adj_hbm, ring_s, sems, i).wait()
        adj = ring_s[jax.lax.rem(i, NRING)]
        rows = pl.ds(i * BI, BI)
        adjb = adj.astype(jnp.bfloat16)
        adj8_s[rows, :] = adjb.astype(jnp.int8)
        c1 = cols1_s[rows, :]
        cond = er1r_s[...] > c1[:, 0:1]
        t = c1[:, 1:2] * F1r_s[...]
        B = jnp.where(cond, t, Fs1r_s[...]) * adjb
        numaug = jnp.dot(B, xaug1_s[...], preferred_element_type=jnp.float32)
        num = numaug[:, :D1]
        denom = numaug[:, D1:D1 + 1]
        recip = 1.0 / jnp.maximum(denom, 1e-12)
        hp = jnp.maximum(num + denom * b1r_ref[...], 0.0)
        x2p = jnp.dot(hp, W2_ref[...], preferred_element_type=jnp.float32)
        lane = jax.lax.broadcasted_iota(jnp.int32, (BI, D2), 1)
        tail = jnp.where(lane == 0, denom, 0.0).astype(jnp.bfloat16)
        xaug2_s[rows, :] = jnp.concatenate(
            [x2p.astype(jnp.bfloat16), tail], axis=1)
        el2, er2 = _scalar_rows(x2p, alar2_ref[...], recip)
        cols2_s[rows, :] = jnp.concatenate(
            [-el2, jnp.exp(0.8 * el2), jnp.zeros((BI, 126), jnp.float32)],
            axis=1).astype(jnp.bfloat16)
        cols = pl.ds(i * BI, BI)
        er2r_s[:, cols] = er2.astype(jnp.bfloat16).reshape(1, BI)
        F2r_s[:, cols] = (jnp.exp(er2) * recip).astype(jnp.bfloat16).reshape(
            1, BI)
        Fs2r_s[:, cols] = (jnp.exp(0.2 * er2) * recip).astype(
            jnp.bfloat16).reshape(1, BI)

    @pl.when(s >= 1 + NB)
    def _att2():
        i = s - 1 - NB
        rows = pl.ds(i * BI, BI)
        adjb = adj8_s[rows, :].astype(jnp.bfloat16)
        c2 = cols2_s[rows, :]
        cond = er2r_s[...] > c2[:, 0:1]
        t = c2[:, 1:2] * F2r_s[...]
        B = jnp.where(cond, t, Fs2r_s[...]) * adjb
        numaug = jnp.dot(B, xaug2_s[...], preferred_element_type=jnp.float32)
        num = numaug[:, :D2]
        d2 = numaug[:, D2:D2 + 1]
        h = num / jnp.maximum(d2, 1e-12) + b2r_ref[...]
        m = jnp.max(h, axis=1, keepdims=True)
        hs = h - m
        lse = jnp.log(jnp.sum(jnp.exp(hs), axis=1, keepdims=True))
        out_ref[...] = hs - lse


def kernel(x, adj, W1, al1, ar1, b1, W2, al2, ar2, b2):
    N, K = x.shape
    D1 = W1.shape[1]
    D2 = W2.shape[1]
    bf = jnp.bfloat16

    return pl.pallas_call(
        _body,
        grid=(1 + 2 * NB,),
        in_specs=[
            pl.BlockSpec((N, K), lambda s: (0, 0)),
            pl.BlockSpec(memory_space=pltpu.MemorySpace.HBM),
            pl.BlockSpec((K, D1), lambda s: (0, 0)),
            pl.BlockSpec((D1, 2), lambda s: (0, 0)),
            pl.BlockSpec((1, D1), lambda s: (0, 0)),
            pl.BlockSpec((D1, D2), lambda s: (0, 0)),
            pl.BlockSpec((D2, 2), lambda s: (0, 0)),
            pl.BlockSpec((1, D2), lambda s: (0, 0)),
        ],
        out_specs=pl.BlockSpec(
            (BI, D2), lambda s: (jnp.clip(s - 1 - NB, 0, NB - 1), 0)),
        out_shape=jax.ShapeDtypeStruct((N, D2), jnp.float32),
        scratch_shapes=[
            pltpu.VMEM((NRING, BI, N), jnp.float32),
            pltpu.VMEM((N, N), jnp.int8),
            pltpu.VMEM((N, 2 * D1), bf),
            pltpu.VMEM((N, 128), bf),
            pltpu.VMEM((1, N), bf),
            pltpu.VMEM((1, N), bf),
            pltpu.VMEM((1, N), bf),
            pltpu.VMEM((N, 2 * D2), bf),
            pltpu.VMEM((N, 128), bf),
            pltpu.VMEM((1, N), bf),
            pltpu.VMEM((1, N), bf),
            pltpu.VMEM((1, N), bf),
            pltpu.SemaphoreType.DMA((NRING,)),
        ],
        compiler_params=pltpu.CompilerParams(
            dimension_semantics=("arbitrary",)),
    )(x, adj, W1, jnp.concatenate([al1, ar1], axis=1),
      b1.reshape(1, D1), W2, jnp.concatenate([al2, ar2], axis=1),
      b2.reshape(1, D2))


# speedup vs baseline: 1.3770x; 1.0275x over previous
import jax
import jax.numpy as jnp
from jax.experimental import pallas as pl
from jax.experimental.pallas import tpu as pltpu

BI = 512
NB = 8
NRING = 3


def _scalar_rows(xp, alar, recip):
    els = jnp.dot(xp, alar, preferred_element_type=jnp.float32)
    el = els[:, 0:1] * recip
    er = els[:, 1:2] * recip
    return el, er


def _adj_copy(adj_hbm, ring_s, sems, blk):
    return pltpu.make_async_copy(
        adj_hbm.at[pl.ds(blk * BI, BI), :],
        ring_s.at[jax.lax.rem(blk, NRING)],
        sems.at[jax.lax.rem(blk, NRING)])


def _body(x_ref, adj_hbm, W1_ref, alar1_ref, b1r_ref,
          W2_ref, alar2_ref, b2r_ref, out_ref,
          ring_s, adj8_s, xaug1_s, nel1_s, rho1_s, er1r_s, F1r_s, Fs1r_s,
          xaug2_s, nel2_s, rho2_s, er2r_s, F2r_s, Fs2r_s, sems):
    s = pl.program_id(0)
    N = adj8_s.shape[1]
    D1 = W1_ref.shape[1]
    D2 = W2_ref.shape[1]

    @pl.when(s == 0)
    def _proj():
        _adj_copy(adj_hbm, ring_s, sems, 0).start()
        _adj_copy(adj_hbm, ring_s, sems, 1).start()
        xv = jnp.dot(x_ref[...].astype(jnp.bfloat16),
                     W1_ref[...].astype(jnp.bfloat16),
                     preferred_element_type=jnp.float32)
        lane = jax.lax.broadcasted_iota(jnp.int32, (N, D1), 1)
        tail = jnp.where(lane == 0, 1.0, 0.0).astype(jnp.bfloat16)
        xaug1_s[...] = jnp.concatenate(
            [xv.astype(jnp.bfloat16), tail], axis=1)
        el, er = _scalar_rows(xv, alar1_ref[...], 1.0)
        nel1_s[...] = (-el).astype(jnp.bfloat16)
        rho1_s[...] = jnp.exp(0.8 * el).astype(jnp.bfloat16)
        er1r_s[...] = er.astype(jnp.bfloat16).reshape(1, N)
        F1r_s[...] = jnp.exp(er).astype(jnp.bfloat16).reshape(1, N)
        Fs1r_s[...] = jnp.exp(0.2 * er).astype(jnp.bfloat16).reshape(1, N)

    @pl.when(jnp.logical_and(s >= 1, s < 1 + NB))
    def _att1():
        i = s - 1

        @pl.when(i + 2 < NB)
        def _prefetch():
            _adj_copy(adj_hbm, ring_s, sems, i + 2).start()

        _adj_copy(adj_hbm, ring_s, sems, i).wait()
        adj = ring_s[jax.lax.rem(i, NRING)]
        rows = pl.ds(i * BI, BI)
        adjb = adj.astype(jnp.bfloat16)
        adj8_s[rows, :] = adjb.astype(jnp.int8)
        cond = er1r_s[...] > nel1_s[rows, :]
        t = rho1_s[rows, :] * F1r_s[...]
        B = jnp.where(cond, t, Fs1r_s[...]) * adjb
        numaug = jnp.dot(B, xaug1_s[...], preferred_element_type=jnp.float32)
        num = numaug[:, :D1]
        denom = numaug[:, D1:D1 + 1]
        recip = 1.0 / jnp.maximum(denom, 1e-12)
        hp = jnp.maximum(num + denom * b1r_ref[...], 0.0)
        x2p = jnp.dot(hp, W2_ref[...], preferred_element_type=jnp.float32)
        lane = jax.lax.broadcasted_iota(jnp.int32, (BI, D2), 1)
        tail = jnp.where(lane == 0, denom, 0.0).astype(jnp.bfloat16)
        xaug2_s[rows, :] = jnp.concatenate(
            [x2p.astype(jnp.bfloat16), tail], axis=1)
        el2, er2 = _scalar_rows(x2p, alar2_ref[...], recip)
        nel2_s[rows, :] = (-el2).astype(jnp.bfloat16)
        rho2_s[rows, :] = jnp.exp(0.8 * el2).astype(jnp.bfloat16)
        cols = pl.ds(i * BI, BI)
        er2r_s[:, cols] = er2.astype(jnp.bfloat16).reshape(1, BI)
        F2r_s[:, cols] = (jnp.exp(er2) * recip).astype(jnp.bfloat16).reshape(
            1, BI)
        Fs2r_s[:, cols] = (jnp.exp(0.2 * er2) * recip).astype(
            jnp.bfloat16).reshape(1, BI)

    @pl.when(s >= 1 + NB)
    def _att2():
        i = s - 1 - NB
        rows = pl.ds(i * BI, BI)
        adjb = adj8_s[rows, :].astype(jnp.bfloat16)
        cond = er2r_s[...] > nel2_s[rows, :]
        t = rho2_s[rows, :] * F2r_s[...]
        B = jnp.where(cond, t, Fs2r_s[...]) * adjb
        numaug = jnp.dot(B, xaug2_s[...], preferred_element_type=jnp.float32)
        num = numaug[:, :D2]
        d2 = numaug[:, D2:D2 + 1]
        h = num / jnp.maximum(d2, 1e-12) + b2r_ref[...]
        m = jnp.max(h, axis=1, keepdims=True)
        hs = h - m
        lse = jnp.log(jnp.sum(jnp.exp(hs), axis=1, keepdims=True))
        out_ref[...] = hs - lse


def kernel(x, adj, W1, al1, ar1, b1, W2, al2, ar2, b2):
    N, K = x.shape
    D1 = W1.shape[1]
    D2 = W2.shape[1]
    bf = jnp.bfloat16

    return pl.pallas_call(
        _body,
        grid=(1 + 2 * NB,),
        in_specs=[
            pl.BlockSpec((N, K), lambda s: (0, 0)),
            pl.BlockSpec(memory_space=pltpu.MemorySpace.HBM),
            pl.BlockSpec((K, D1), lambda s: (0, 0)),
            pl.BlockSpec((D1, 2), lambda s: (0, 0)),
            pl.BlockSpec((1, D1), lambda s: (0, 0)),
            pl.BlockSpec((D1, D2), lambda s: (0, 0)),
            pl.BlockSpec((D2, 2), lambda s: (0, 0)),
            pl.BlockSpec((1, D2), lambda s: (0, 0)),
        ],
        out_specs=pl.BlockSpec(
            (BI, D2), lambda s: (jnp.clip(s - 1 - NB, 0, NB - 1), 0)),
        out_shape=jax.ShapeDtypeStruct((N, D2), jnp.float32),
        scratch_shapes=[
            pltpu.VMEM((NRING, BI, N), jnp.float32),
            pltpu.VMEM((N, N), jnp.int8),
            pltpu.VMEM((N, 2 * D1), bf),
            pltpu.VMEM((N, 1), bf),
            pltpu.VMEM((N, 1), bf),
            pltpu.VMEM((1, N), bf),
            pltpu.VMEM((1, N), bf),
            pltpu.VMEM((1, N), bf),
            pltpu.VMEM((N, 2 * D2), bf),
            pltpu.VMEM((N, 1), bf),
            pltpu.VMEM((N, 1), bf),
            pltpu.VMEM((1, N), bf),
            pltpu.VMEM((1, N), bf),
            pltpu.VMEM((1, N), bf),
            pltpu.SemaphoreType.DMA((NRING,)),
        ],
        compiler_params=pltpu.CompilerParams(
            dimension_semantics=("arbitrary",)),
    )(x, adj, W1, jnp.concatenate([al1, ar1], axis=1),
      b1.reshape(1, D1), W2, jnp.concatenate([al2, ar2], axis=1),
      b2.reshape(1, D2))
